# Initial kernel scaffold; baseline (speedup 1.0000x reference)
import functools

import jax
import jax.numpy as jnp
from jax import lax
from jax.experimental import pallas as pl
from jax.experimental.pallas import tpu as pltpu
from jax.experimental.pallas import tpu_sc as plsc

N = 10000
E = 160000
D_IN = 256
D_H = 256
D_OUT = 64

NC = 2
NS = 16
LANE = 128

EP = 163840
CH = EP // (NS * LANE)
ACC_ROWS = 10240
ROWS_PER_SUB = ACC_ROWS // NS
DUMMY_ROW = N

_HIGH = lax.Precision.HIGHEST


def _make_segsum(d, with_deg):
    mesh = plsc.VectorSubcoreMesh(core_axis_name="c", subcore_axis_name="s")

    out_type = [jax.ShapeDtypeStruct((NC * ACC_ROWS, d), jnp.float32)]
    scratch = [
        pltpu.VMEM((CH, LANE), jnp.int32),
        pltpu.VMEM((CH, LANE), jnp.int32),
        pltpu.VMEM((LANE, d), jnp.float32),
        pltpu.VMEM_SHARED((ACC_ROWS, d), jnp.float32),
    ]
    if with_deg:
        out_type.append(jax.ShapeDtypeStruct((NC * ACC_ROWS, 16), jnp.float32))
        scratch += [
            pltpu.VMEM((LANE, 16), jnp.float32),
            pltpu.VMEM_SHARED((ACC_ROWS, 16), jnp.float32),
        ]

    def body(h0_hbm, h1_hbm, src_hbm, dst_hbm, z_hbm, z16_hbm, ones_hbm,
             *rest):
        if with_deg:
            ns_out, deg_out, src_v, dst_v, rowbuf, acc, onesbuf, dacc = rest
        else:
            ns_out, src_v, dst_v, rowbuf, acc = rest
        cid = lax.axis_index("c")
        sid = lax.axis_index("s")
        row0 = sid * ROWS_PER_SUB

        pltpu.sync_copy(src_hbm.at[sid], src_v)
        pltpu.sync_copy(dst_hbm.at[sid], dst_v)
        pltpu.sync_copy(z_hbm.at[pl.ds(row0, ROWS_PER_SUB)],
                        acc.at[pl.ds(row0, ROWS_PER_SUB)])
        if with_deg:
            pltpu.sync_copy(z16_hbm.at[pl.ds(row0, ROWS_PER_SUB)],
                            dacc.at[pl.ds(row0, ROWS_PER_SUB)])
            pltpu.sync_copy(ones_hbm, onesbuf)
        plsc.subcore_barrier()

        def edge_loop(h_ref):
            @pl.loop(0, CH)
            def _(j):
                pltpu.sync_copy(h_ref.at[src_v.at[j]], rowbuf)
                pltpu.sync_copy(rowbuf, acc.at[dst_v.at[j]], add=True)
                if with_deg:
                    pltpu.sync_copy(onesbuf, dacc.at[dst_v.at[j]], add=True)

        @pl.when(cid == 0)
        def _():
            edge_loop(h0_hbm)

        @pl.when(cid == 1)
        def _():
            edge_loop(h1_hbm)

        plsc.subcore_barrier()

        base = cid * ACC_ROWS + row0
        pltpu.sync_copy(acc.at[pl.ds(row0, ROWS_PER_SUB)],
                        ns_out.at[pl.ds(base, ROWS_PER_SUB)])
        if with_deg:
            pltpu.sync_copy(dacc.at[pl.ds(row0, ROWS_PER_SUB)],
                            deg_out.at[pl.ds(base, ROWS_PER_SUB)])

    return functools.partial(
        pl.kernel, mesh=mesh, out_type=out_type, scratch_types=scratch,
    )(body)


_segsum_wide_deg = _make_segsum(128, True)
_segsum_wide = _make_segsum(128, False)
_segsum_narrow = _make_segsum(32, False)


_BN = 1000
_GRID = (N // _BN,)


def _row_spec(w):
    return pl.BlockSpec((_BN, w), lambda i: (i, 0))


def _half_spec(half, w):
    return pl.BlockSpec((1, _BN, w), lambda i, h=half: (h, i, 0))


def _full_spec(*shape):
    ndim = len(shape)
    return pl.BlockSpec(shape, lambda i: (0,) * ndim)


def _scale_from_deg(dg_ref):
    return 1.0 / jnp.clip(dg_ref[0, :, 0:1], 1.0, None)


def _combine_body(h0, h1, ns0, ns1, dg, wst, wnt, b, o0, o1):
    scale = _scale_from_deg(dg)
    acc = (
        jnp.dot(h0[...], wst[0:128, :], precision=_HIGH)
        + jnp.dot(h1[...], wst[128:256, :], precision=_HIGH)
        + jnp.dot(ns0[0] * scale, wnt[0:128, :], precision=_HIGH)
        + jnp.dot(ns1[0] * scale, wnt[128:256, :], precision=_HIGH)
        + b[...]
    )
    acc = jnp.maximum(acc, 0.0)
    o0[...] = acc[:, 0:128]
    o1[...] = acc[:, 128:256]


def _combine_pre_body(h0, h1, ns0, ns1, dg, wst, wnt, b, wpre, o0, o1, g0, g1):
    scale = _scale_from_deg(dg)
    acc = (
        jnp.dot(h0[...], wst[0:128, :], precision=_HIGH)
        + jnp.dot(h1[...], wst[128:256, :], precision=_HIGH)
        + jnp.dot(ns0[0] * scale, wnt[0:128, :], precision=_HIGH)
        + jnp.dot(ns1[0] * scale, wnt[128:256, :], precision=_HIGH)
        + b[...]
    )
    acc = jnp.maximum(acc, 0.0)
    o0[...] = acc[:, 0:128]
    o1[...] = acc[:, 128:256]
    g = jnp.dot(acc, wpre[...], precision=_HIGH)
    g0[...] = g[:, 0:32]
    g1[...] = g[:, 32:64]


def _final_body(h0, h1, ns0, ns1, dg, wst, b, o):
    scale = _scale_from_deg(dg)
    hn = jnp.concatenate([ns0[0], ns1[0]], axis=1) * scale
    o[...] = (
        jnp.dot(h0[...], wst[0:128, :], precision=_HIGH)
        + jnp.dot(h1[...], wst[128:256, :], precision=_HIGH)
        + hn
        + b[...]
    )


def _combine(h0, h1, ns, dg, wst, wnt, b):
    return pl.pallas_call(
        _combine_body,
        grid=_GRID,
        in_specs=[
            _row_spec(128), _row_spec(128),
            _half_spec(0, 128), _half_spec(1, 128),
            _half_spec(0, 16),
            _full_spec(256, 256), _full_spec(256, 256), _full_spec(1, 256),
        ],
        out_specs=[_row_spec(128), _row_spec(128)],
        out_shape=[jax.ShapeDtypeStruct((N, 128), jnp.float32)] * 2,
    )(h0, h1, ns, ns, dg, wst, wnt, b)


def _combine_pre(h0, h1, ns, dg, wst, wnt, b, wpre):
    return pl.pallas_call(
        _combine_pre_body,
        grid=_GRID,
        in_specs=[
            _row_spec(128), _row_spec(128),
            _half_spec(0, 128), _half_spec(1, 128),
            _half_spec(0, 16),
            _full_spec(256, 256), _full_spec(256, 256), _full_spec(1, 256),
            _full_spec(256, 64),
        ],
        out_specs=[_row_spec(128), _row_spec(128),
                   _row_spec(32), _row_spec(32)],
        out_shape=[
            jax.ShapeDtypeStruct((N, 128), jnp.float32),
            jax.ShapeDtypeStruct((N, 128), jnp.float32),
            jax.ShapeDtypeStruct((N, 32), jnp.float32),
            jax.ShapeDtypeStruct((N, 32), jnp.float32),
        ],
    )(h0, h1, ns, ns, dg, wst, wnt, b, wpre)


def _final(h0, h1, ns, dg, wst, b):
    return pl.pallas_call(
        _final_body,
        grid=_GRID,
        in_specs=[
            _row_spec(128), _row_spec(128),
            _half_spec(0, 32), _half_spec(1, 32),
            _half_spec(0, 16),
            _full_spec(256, 64), _full_spec(1, 64),
        ],
        out_specs=_row_spec(64),
        out_shape=jax.ShapeDtypeStruct((N, 64), jnp.float32),
    )(h0, h1, ns, ns, dg, wst, b)


def kernel(x, edge_index, W_neigh0, W_self0, b0, W_neigh1, W_self1, b1,
           W_neigh2, W_self2, b2):
    src = edge_index[0]
    dst = edge_index[1]
    srcp = jnp.concatenate(
        [src, jnp.zeros((EP - E,), jnp.int32)]).reshape(NS, CH, LANE)
    dstp = jnp.concatenate(
        [dst, jnp.full((EP - E,), DUMMY_ROW, jnp.int32)]).reshape(NS, CH, LANE)

    zeros_w = jnp.zeros((ACC_ROWS, 128), jnp.float32)
    zeros_n = jnp.zeros((ACC_ROWS, 32), jnp.float32)
    zeros16 = jnp.zeros((ACC_ROWS, 16), jnp.float32)
    ones16 = jnp.ones((LANE, 16), jnp.float32)

    x0 = x[:, 0:128]
    x1 = x[:, 128:256]

    wst0 = W_self0.T
    wnt0 = W_neigh0.T
    wst1 = W_self1.T
    wnt1 = W_neigh1.T
    wnt2 = W_neigh2.T
    wst2 = W_self2.T
    b0r = b0.reshape(1, D_H)
    b1r = b1.reshape(1, D_H)
    b2r = b2.reshape(1, D_OUT)

    ns0_f, deg_f = _segsum_wide_deg(x0, x1, srcp, dstp, zeros_w, zeros16,
                                    ones16)
    ns0 = ns0_f.reshape(NC, ACC_ROWS, 128)
    dg = deg_f.reshape(NC, ACC_ROWS, 16)

    h10, h11 = _combine(x0, x1, ns0, dg, wst0, wnt0, b0r)

    ns1_f = _segsum_wide(h10, h11, srcp, dstp, zeros_w, zeros16, ones16)
    ns1 = ns1_f.reshape(NC, ACC_ROWS, 128)

    h20, h21, g0, g1 = _combine_pre(h10, h11, ns1, dg, wst1, wnt1, b1r, wnt2)

    ns2_f = _segsum_narrow(g0, g1, srcp, dstp, zeros_n, zeros16, ones16)
    ns2 = ns2_f.reshape(NC, ACC_ROWS, 32)

    return _final(h20, h21, ns2, dg, wst2, b2r)


# speedup vs baseline: 2.5792x; 2.5792x over previous
import functools

import jax
import jax.numpy as jnp
from jax import lax
from jax.experimental import pallas as pl
from jax.experimental.pallas import tpu as pltpu
from jax.experimental.pallas import tpu_sc as plsc

N = 10000
E = 160000
D_IN = 256
D_H = 256
D_OUT = 64

NC = 2
NS = 16
LANE = 128

EP = 163840
CH = EP // (NS * LANE)
ACC_ROWS = 10240
ROWS_PER_SUB = ACC_ROWS // NS
DUMMY_ROW = N
KI = 8
NB = CH // KI
CHD = CH // NC

_HIGH = lax.Precision.HIGHEST


_MESH = plsc.VectorSubcoreMesh(core_axis_name="c", subcore_axis_name="s")


def _make_segsum():
    out_type = [jax.ShapeDtypeStruct((NC * ACC_ROWS, 128), jnp.float32)]
    scratch = [
        pltpu.VMEM((KI, LANE), jnp.int32),
        pltpu.VMEM((KI, LANE), jnp.int32),
        pltpu.VMEM((LANE, 128), jnp.float32),
        pltpu.VMEM_SHARED((ACC_ROWS, 128), jnp.float32),
    ]

    def body(h0_hbm, h1_hbm, src_hbm, dst_hbm, z_hbm,
             ns_out, src_v, dst_v, rowbuf, acc):
        cid = lax.axis_index("c")
        sid = lax.axis_index("s")
        row0 = sid * ROWS_PER_SUB

        pltpu.sync_copy(z_hbm.at[pl.ds(row0, ROWS_PER_SUB)],
                        acc.at[pl.ds(row0, ROWS_PER_SUB)])
        plsc.subcore_barrier()

        def edge_loop(h_ref):
            @pl.loop(0, NB)
            def _(blk):
                pltpu.sync_copy(src_hbm.at[sid, pl.ds(blk * KI, KI)], src_v)
                pltpu.sync_copy(dst_hbm.at[sid, pl.ds(blk * KI, KI)], dst_v)

                @pl.loop(0, KI)
                def _(j):
                    pltpu.sync_copy(h_ref.at[src_v.at[j]], rowbuf)
                    pltpu.sync_copy(rowbuf, acc.at[dst_v.at[j]], add=True)

        @pl.when(cid == 0)
        def _():
            edge_loop(h0_hbm)

        @pl.when(cid == 1)
        def _():
            edge_loop(h1_hbm)

        plsc.subcore_barrier()

        base = cid * ACC_ROWS + row0
        pltpu.sync_copy(acc.at[pl.ds(row0, ROWS_PER_SUB)],
                        ns_out.at[pl.ds(base, ROWS_PER_SUB)])

    return functools.partial(
        pl.kernel, mesh=_MESH, out_type=out_type, scratch_types=scratch,
    )(body)


def _make_deg():
    out_type = [jax.ShapeDtypeStruct((NC * ACC_ROWS, 128), jnp.float32)]
    scratch = [
        pltpu.VMEM((KI, LANE), jnp.int32),
        pltpu.VMEM((LANE, 128), jnp.float32),
        pltpu.VMEM_SHARED((ACC_ROWS, 128), jnp.float32),
    ]

    def body(dst_hbm, z_hbm, ones_hbm, deg_out, dst_v, onesbuf, dacc):
        cid = lax.axis_index("c")
        sid = lax.axis_index("s")
        row0 = sid * ROWS_PER_SUB

        pltpu.sync_copy(z_hbm.at[pl.ds(row0, ROWS_PER_SUB)],
                        dacc.at[pl.ds(row0, ROWS_PER_SUB)])
        pltpu.sync_copy(ones_hbm, onesbuf)
        plsc.subcore_barrier()

        @pl.loop(0, CHD // KI)
        def _(blk):
            pltpu.sync_copy(
                dst_hbm.at[sid, pl.ds(cid * CHD + blk * KI, KI)], dst_v)

            @pl.loop(0, KI)
            def _(j):
                pltpu.sync_copy(onesbuf, dacc.at[dst_v.at[j]], add=True)

        plsc.subcore_barrier()

        base = cid * ACC_ROWS + row0
        pltpu.sync_copy(dacc.at[pl.ds(row0, ROWS_PER_SUB)],
                        deg_out.at[pl.ds(base, ROWS_PER_SUB)])

    return functools.partial(
        pl.kernel, mesh=_MESH, out_type=out_type, scratch_types=scratch,
    )(body)


_segsum = _make_segsum()
_deg = _make_deg()


_BN = 1000
_GRID = (N // _BN,)


def _row_spec(w):
    return pl.BlockSpec((_BN, w), lambda i: (i, 0))


def _half_spec(half, w):
    return pl.BlockSpec((1, _BN, w), lambda i, h=half: (h, i, 0))


def _full_spec(*shape):
    ndim = len(shape)
    return pl.BlockSpec(shape, lambda i: (0,) * ndim)


def _scale_from_deg(dg0, dg1):
    return 1.0 / jnp.clip(dg0[0, :, 0:1] + dg1[0, :, 0:1], 1.0, None)


def _combine_body(h0, h1, ns0, ns1, dg0, dg1, wst, wnt, b, o0, o1):
    scale = _scale_from_deg(dg0, dg1)
    acc = (
        jnp.dot(h0[...], wst[0:128, :], precision=_HIGH)
        + jnp.dot(h1[...], wst[128:256, :], precision=_HIGH)
        + jnp.dot(ns0[0] * scale, wnt[0:128, :], precision=_HIGH)
        + jnp.dot(ns1[0] * scale, wnt[128:256, :], precision=_HIGH)
        + b[...]
    )
    acc = jnp.maximum(acc, 0.0)
    o0[...] = acc[:, 0:128]
    o1[...] = acc[:, 128:256]


def _final_body(h0, h1, ns0, ns1, dg0, dg1, wst, wnt, b, o):
    scale = _scale_from_deg(dg0, dg1)
    o[...] = (
        jnp.dot(h0[...], wst[0:128, :], precision=_HIGH)
        + jnp.dot(h1[...], wst[128:256, :], precision=_HIGH)
        + jnp.dot(ns0[0] * scale, wnt[0:128, :], precision=_HIGH)
        + jnp.dot(ns1[0] * scale, wnt[128:256, :], precision=_HIGH)
        + b[...]
    )


def _combine(h0, h1, ns, dg, wst, wnt, b):
    return pl.pallas_call(
        _combine_body,
        grid=_GRID,
        in_specs=[
            _row_spec(128), _row_spec(128),
            _half_spec(0, 128), _half_spec(1, 128),
            _half_spec(0, 128), _half_spec(1, 128),
            _full_spec(256, 256), _full_spec(256, 256), _full_spec(1, 256),
        ],
        out_specs=[_row_spec(128), _row_spec(128)],
        out_shape=[jax.ShapeDtypeStruct((N, 128), jnp.float32)] * 2,
    )(h0, h1, ns, ns, dg, dg, wst, wnt, b)


def _final(h0, h1, ns, dg, wst, wnt, b):
    return pl.pallas_call(
        _final_body,
        grid=_GRID,
        in_specs=[
            _row_spec(128), _row_spec(128),
            _half_spec(0, 128), _half_spec(1, 128),
            _half_spec(0, 128), _half_spec(1, 128),
            _full_spec(256, 64), _full_spec(256, 64), _full_spec(1, 64),
        ],
        out_specs=_row_spec(64),
        out_shape=jax.ShapeDtypeStruct((N, 64), jnp.float32),
    )(h0, h1, ns, ns, dg, dg, wst, wnt, b)


def kernel(x, edge_index, W_neigh0, W_self0, b0, W_neigh1, W_self1, b1,
           W_neigh2, W_self2, b2):
    src = edge_index[0]
    dst = edge_index[1]
    srcp = jnp.concatenate(
        [src, jnp.zeros((EP - E,), jnp.int32)]).reshape(NS, CH, LANE)
    dstp = jnp.concatenate(
        [dst, jnp.full((EP - E,), DUMMY_ROW, jnp.int32)]).reshape(NS, CH, LANE)

    zeros_w = jnp.zeros((ACC_ROWS, 128), jnp.float32)
    ones_w = jnp.ones((LANE, 128), jnp.float32)

    x0 = x[:, 0:128]
    x1 = x[:, 128:256]

    wst0 = W_self0.T
    wnt0 = W_neigh0.T
    wst1 = W_self1.T
    wnt1 = W_neigh1.T
    wnt2 = W_neigh2.T
    wst2 = W_self2.T
    b0r = b0.reshape(1, D_H)
    b1r = b1.reshape(1, D_H)
    b2r = b2.reshape(1, D_OUT)

    deg_f, = _deg(dstp, zeros_w, ones_w)
    dg = deg_f.reshape(NC, ACC_ROWS, 128)

    ns0_f, = _segsum(x0, x1, srcp, dstp, zeros_w)
    ns0 = ns0_f.reshape(NC, ACC_ROWS, 128)

    h10, h11 = _combine(x0, x1, ns0, dg, wst0, wnt0, b0r)

    ns1_f, = _segsum(h10, h11, srcp, dstp, zeros_w)
    ns1 = ns1_f.reshape(NC, ACC_ROWS, 128)

    h20, h21 = _combine(h10, h11, ns1, dg, wst1, wnt1, b1r)

    ns2_f, = _segsum(h20, h21, srcp, dstp, zeros_w)
    ns2 = ns2_f.reshape(NC, ACC_ROWS, 128)

    return _final(h20, h21, ns2, dg, wst2, wnt2, b2r)


# speedup vs baseline: 2.8417x; 1.1018x over previous
import functools

import jax
import jax.numpy as jnp
from jax import lax
from jax.experimental import pallas as pl
from jax.experimental.pallas import tpu as pltpu
from jax.experimental.pallas import tpu_sc as plsc

N = 10000
E = 160000
D_IN = 256
D_H = 256
D_OUT = 64

NC = 2
NS = 16
LANE = 128

EP = 163840
CH = EP // (NS * LANE)
ACC_ROWS = 10240
ROWS_PER_SUB = ACC_ROWS // NS
DUMMY_ROW = N
KI = 8
NB = CH // KI
CHD = CH // NC

_HIGH = lax.Precision.HIGHEST


_MESH = plsc.VectorSubcoreMesh(core_axis_name="c", subcore_axis_name="s")


def _make_segsum():
    out_type = [jax.ShapeDtypeStruct((NC * ACC_ROWS, 128), jnp.float32)]
    scratch = [
        pltpu.VMEM((KI, LANE), jnp.int32),
        pltpu.VMEM((KI, LANE), jnp.int32),
        pltpu.VMEM((LANE, 128), jnp.float32),
        pltpu.VMEM((LANE, 128), jnp.float32),
        pltpu.SemaphoreType.DMA,
        pltpu.SemaphoreType.DMA,
        pltpu.VMEM_SHARED((ACC_ROWS, 128), jnp.float32),
    ]

    def body(h0_hbm, h1_hbm, src_hbm, dst_hbm, z_hbm,
             ns_out, src_v, dst_v, buf_a, buf_b, sem_a, sem_b, acc):
        cid = lax.axis_index("c")
        sid = lax.axis_index("s")
        row0 = sid * ROWS_PER_SUB
        bufs = (buf_a, buf_b)
        sems = (sem_a, sem_b)

        pltpu.sync_copy(z_hbm.at[pl.ds(row0, ROWS_PER_SUB)],
                        acc.at[pl.ds(row0, ROWS_PER_SUB)])
        plsc.subcore_barrier()

        def edge_loop(h_ref):
            @pl.loop(0, NB)
            def _(blk):
                pltpu.sync_copy(src_hbm.at[sid, pl.ds(blk * KI, KI)], src_v)
                pltpu.sync_copy(dst_hbm.at[sid, pl.ds(blk * KI, KI)], dst_v)

                cp = pltpu.async_copy(h_ref.at[src_v.at[0]], buf_a, sem_a)
                for j in range(KI):
                    cp.wait()
                    if j + 1 < KI:
                        cp = pltpu.async_copy(h_ref.at[src_v.at[j + 1]],
                                              bufs[(j + 1) % 2],
                                              sems[(j + 1) % 2])
                    pltpu.sync_copy(bufs[j % 2], acc.at[dst_v.at[j]],
                                    add=True)

        @pl.when(cid == 0)
        def _():
            edge_loop(h0_hbm)

        @pl.when(cid == 1)
        def _():
            edge_loop(h1_hbm)

        plsc.subcore_barrier()

        base = cid * ACC_ROWS + row0
        pltpu.sync_copy(acc.at[pl.ds(row0, ROWS_PER_SUB)],
                        ns_out.at[pl.ds(base, ROWS_PER_SUB)])

    return functools.partial(
        pl.kernel, mesh=_MESH, out_type=out_type, scratch_types=scratch,
    )(body)


def _make_deg():
    out_type = [jax.ShapeDtypeStruct((NC * ACC_ROWS, 128), jnp.float32)]
    scratch = [
        pltpu.VMEM((KI, LANE), jnp.int32),
        pltpu.VMEM((LANE, 128), jnp.float32),
        pltpu.VMEM_SHARED((ACC_ROWS, 128), jnp.float32),
    ]

    def body(dst_hbm, z_hbm, ones_hbm, deg_out, dst_v, onesbuf, dacc):
        cid = lax.axis_index("c")
        sid = lax.axis_index("s")
        row0 = sid * ROWS_PER_SUB

        pltpu.sync_copy(z_hbm.at[pl.ds(row0, ROWS_PER_SUB)],
                        dacc.at[pl.ds(row0, ROWS_PER_SUB)])
        pltpu.sync_copy(ones_hbm, onesbuf)
        plsc.subcore_barrier()

        @pl.loop(0, CHD // KI)
        def _(blk):
            pltpu.sync_copy(
                dst_hbm.at[sid, pl.ds(cid * CHD + blk * KI, KI)], dst_v)

            @pl.loop(0, KI)
            def _(j):
                pltpu.sync_copy(onesbuf, dacc.at[dst_v.at[j]], add=True)

        plsc.subcore_barrier()

        base = cid * ACC_ROWS + row0
        pltpu.sync_copy(dacc.at[pl.ds(row0, ROWS_PER_SUB)],
                        deg_out.at[pl.ds(base, ROWS_PER_SUB)])

    return functools.partial(
        pl.kernel, mesh=_MESH, out_type=out_type, scratch_types=scratch,
    )(body)


_segsum = _make_segsum()
_deg = _make_deg()


_BN = 1000
_GRID = (N // _BN,)


def _row_spec(w):
    return pl.BlockSpec((_BN, w), lambda i: (i, 0))


def _half_spec(half, w):
    return pl.BlockSpec((1, _BN, w), lambda i, h=half: (h, i, 0))


def _full_spec(*shape):
    ndim = len(shape)
    return pl.BlockSpec(shape, lambda i: (0,) * ndim)


def _scale_from_deg(dg0, dg1):
    return 1.0 / jnp.clip(dg0[0, :, 0:1] + dg1[0, :, 0:1], 1.0, None)


def _combine_body(h0, h1, ns0, ns1, dg0, dg1, wst, wnt, b, o0, o1):
    scale = _scale_from_deg(dg0, dg1)
    acc = (
        jnp.dot(h0[...], wst[0:128, :], precision=_HIGH)
        + jnp.dot(h1[...], wst[128:256, :], precision=_HIGH)
        + jnp.dot(ns0[0] * scale, wnt[0:128, :], precision=_HIGH)
        + jnp.dot(ns1[0] * scale, wnt[128:256, :], precision=_HIGH)
        + b[...]
    )
    acc = jnp.maximum(acc, 0.0)
    o0[...] = acc[:, 0:128]
    o1[...] = acc[:, 128:256]


def _final_body(h0, h1, ns0, ns1, dg0, dg1, wst, wnt, b, o):
    scale = _scale_from_deg(dg0, dg1)
    o[...] = (
        jnp.dot(h0[...], wst[0:128, :], precision=_HIGH)
        + jnp.dot(h1[...], wst[128:256, :], precision=_HIGH)
        + jnp.dot(ns0[0] * scale, wnt[0:128, :], precision=_HIGH)
        + jnp.dot(ns1[0] * scale, wnt[128:256, :], precision=_HIGH)
        + b[...]
    )


def _combine(h0, h1, ns, dg, wst, wnt, b):
    return pl.pallas_call(
        _combine_body,
        grid=_GRID,
        in_specs=[
            _row_spec(128), _row_spec(128),
            _half_spec(0, 128), _half_spec(1, 128),
            _half_spec(0, 128), _half_spec(1, 128),
            _full_spec(256, 256), _full_spec(256, 256), _full_spec(1, 256),
        ],
        out_specs=[_row_spec(128), _row_spec(128)],
        out_shape=[jax.ShapeDtypeStruct((N, 128), jnp.float32)] * 2,
    )(h0, h1, ns, ns, dg, dg, wst, wnt, b)


def _final(h0, h1, ns, dg, wst, wnt, b):
    return pl.pallas_call(
        _final_body,
        grid=_GRID,
        in_specs=[
            _row_spec(128), _row_spec(128),
            _half_spec(0, 128), _half_spec(1, 128),
            _half_spec(0, 128), _half_spec(1, 128),
            _full_spec(256, 64), _full_spec(256, 64), _full_spec(1, 64),
        ],
        out_specs=_row_spec(64),
        out_shape=jax.ShapeDtypeStruct((N, 64), jnp.float32),
    )(h0, h1, ns, ns, dg, dg, wst, wnt, b)


def kernel(x, edge_index, W_neigh0, W_self0, b0, W_neigh1, W_self1, b1,
           W_neigh2, W_self2, b2):
    src = edge_index[0]
    dst = edge_index[1]
    srcp = jnp.concatenate(
        [src, jnp.zeros((EP - E,), jnp.int32)]).reshape(NS, CH, LANE)
    dstp = jnp.concatenate(
        [dst, jnp.full((EP - E,), DUMMY_ROW, jnp.int32)]).reshape(NS, CH, LANE)

    zeros_w = jnp.zeros((ACC_ROWS, 128), jnp.float32)
    ones_w = jnp.ones((LANE, 128), jnp.float32)

    x0 = x[:, 0:128]
    x1 = x[:, 128:256]

    wst0 = W_self0.T
    wnt0 = W_neigh0.T
    wst1 = W_self1.T
    wnt1 = W_neigh1.T
    wnt2 = W_neigh2.T
    wst2 = W_self2.T
    b0r = b0.reshape(1, D_H)
    b1r = b1.reshape(1, D_H)
    b2r = b2.reshape(1, D_OUT)

    deg_f, = _deg(dstp, zeros_w, ones_w)
    dg = deg_f.reshape(NC, ACC_ROWS, 128)

    ns0_f, = _segsum(x0, x1, srcp, dstp, zeros_w)
    ns0 = ns0_f.reshape(NC, ACC_ROWS, 128)

    h10, h11 = _combine(x0, x1, ns0, dg, wst0, wnt0, b0r)

    ns1_f, = _segsum(h10, h11, srcp, dstp, zeros_w)
    ns1 = ns1_f.reshape(NC, ACC_ROWS, 128)

    h20, h21 = _combine(h10, h11, ns1, dg, wst1, wnt1, b1r)

    ns2_f, = _segsum(h20, h21, srcp, dstp, zeros_w)
    ns2 = ns2_f.reshape(NC, ACC_ROWS, 128)

    return _final(h20, h21, ns2, dg, wst2, wnt2, b2r)


# speedup vs baseline: 5.2572x; 1.8500x over previous
import functools

import jax
import jax.numpy as jnp
from jax import lax
from jax.experimental import pallas as pl
from jax.experimental.pallas import tpu as pltpu
from jax.experimental.pallas import tpu_sc as plsc

N = 10000
E = 160000
D_IN = 256
D_H = 256
D_OUT = 64

NC = 2
NS = 16
LANE = 128

EP = 163840
CH = EP // (NS * LANE)
ACC_ROWS = 10240
ROWS_PER_SUB = ACC_ROWS // NS
DUMMY_ROW = N
KI = 8
NB = CH // KI
CHD = CH // NC

_HIGH = lax.Precision.HIGHEST


_MESH = plsc.VectorSubcoreMesh(core_axis_name="c", subcore_axis_name="s")


def _make_segsum():
    out_type = [jax.ShapeDtypeStruct((NC * ACC_ROWS, 128), jnp.float32)]
    scratch = [
        pltpu.VMEM((KI, LANE), jnp.int32),
        pltpu.VMEM((KI, LANE), jnp.int32),
        pltpu.VMEM((LANE, 128), jnp.float32),
        pltpu.VMEM((LANE, 128), jnp.float32),
        pltpu.SemaphoreType.DMA,
        pltpu.SemaphoreType.DMA,
        pltpu.VMEM_SHARED((ACC_ROWS, 128), jnp.float32),
    ]

    def body(h0_hbm, h1_hbm, src_hbm, dst_hbm, z_hbm,
             ns_out, src_v, dst_v, buf_a, buf_b, sem_a, sem_b, acc):
        cid = lax.axis_index("c")
        sid = lax.axis_index("s")
        row0 = sid * ROWS_PER_SUB
        bufs = (buf_a, buf_b)
        sems = (sem_a, sem_b)

        pltpu.sync_copy(z_hbm.at[pl.ds(row0, ROWS_PER_SUB)],
                        acc.at[pl.ds(row0, ROWS_PER_SUB)])
        plsc.subcore_barrier()

        def edge_loop(h_ref):
            @pl.loop(0, NB)
            def _(blk):
                pltpu.sync_copy(src_hbm.at[sid, pl.ds(blk * KI, KI)], src_v)
                pltpu.sync_copy(dst_hbm.at[sid, pl.ds(blk * KI, KI)], dst_v)

                cp = pltpu.async_copy(h_ref.at[src_v.at[0]], buf_a, sem_a)
                for j in range(KI):
                    cp.wait()
                    if j + 1 < KI:
                        cp = pltpu.async_copy(h_ref.at[src_v.at[j + 1]],
                                              bufs[(j + 1) % 2],
                                              sems[(j + 1) % 2])
                    pltpu.sync_copy(bufs[j % 2], acc.at[dst_v.at[j]],
                                    add=True)

        @pl.when(cid == 0)
        def _():
            edge_loop(h0_hbm)

        @pl.when(cid == 1)
        def _():
            edge_loop(h1_hbm)

        plsc.subcore_barrier()

        base = cid * ACC_ROWS + row0
        pltpu.sync_copy(acc.at[pl.ds(row0, ROWS_PER_SUB)],
                        ns_out.at[pl.ds(base, ROWS_PER_SUB)])

    return functools.partial(
        pl.kernel, mesh=_MESH, out_type=out_type, scratch_types=scratch,
    )(body)


def _make_deg():
    out_type = [jax.ShapeDtypeStruct((NC * ACC_ROWS, 128), jnp.float32)]
    scratch = [
        pltpu.VMEM((KI, LANE), jnp.int32),
        pltpu.VMEM((LANE, 128), jnp.float32),
        pltpu.VMEM_SHARED((ACC_ROWS, 128), jnp.float32),
    ]

    def body(dst_hbm, z_hbm, ones_hbm, deg_out, dst_v, onesbuf, dacc):
        cid = lax.axis_index("c")
        sid = lax.axis_index("s")
        row0 = sid * ROWS_PER_SUB

        pltpu.sync_copy(z_hbm.at[pl.ds(row0, ROWS_PER_SUB)],
                        dacc.at[pl.ds(row0, ROWS_PER_SUB)])
        pltpu.sync_copy(ones_hbm, onesbuf)
        plsc.subcore_barrier()

        @pl.loop(0, CHD // KI)
        def _(blk):
            pltpu.sync_copy(
                dst_hbm.at[sid, pl.ds(cid * CHD + blk * KI, KI)], dst_v)

            @pl.loop(0, KI)
            def _(j):
                pltpu.sync_copy(onesbuf, dacc.at[dst_v.at[j]], add=True)

        plsc.subcore_barrier()

        base = cid * ACC_ROWS + row0
        pltpu.sync_copy(dacc.at[pl.ds(row0, ROWS_PER_SUB)],
                        deg_out.at[pl.ds(base, ROWS_PER_SUB)])

    return functools.partial(
        pl.kernel, mesh=_MESH, out_type=out_type, scratch_types=scratch,
    )(body)


_segsum = _make_segsum()
_deg = _make_deg()


_BN = 1000
_GRID = (N // _BN,)


def _row_spec(w):
    return pl.BlockSpec((_BN, w), lambda i: (i, 0))


def _half_spec(half, w):
    return pl.BlockSpec((1, _BN, w), lambda i, h=half: (h, i, 0))


def _full_spec(*shape):
    ndim = len(shape)
    return pl.BlockSpec(shape, lambda i: (0,) * ndim)


def _scale_from_deg(dg0, dg1):
    return 1.0 / jnp.clip(dg0[0, :, 0:1] + dg1[0, :, 0:1], 1.0, None)


def _combine_body(h0, h1, ns0, ns1, dg0, dg1, wst, wnt, b, o0, o1):
    scale = _scale_from_deg(dg0, dg1)
    acc = (
        jnp.dot(h0[...], wst[0:128, :], precision=_HIGH)
        + jnp.dot(h1[...], wst[128:256, :], precision=_HIGH)
        + jnp.dot(ns0[0] * scale, wnt[0:128, :], precision=_HIGH)
        + jnp.dot(ns1[0] * scale, wnt[128:256, :], precision=_HIGH)
        + b[...]
    )
    acc = jnp.maximum(acc, 0.0)
    o0[...] = acc[:, 0:128]
    o1[...] = acc[:, 128:256]


def _final_body(h0, h1, ns0, ns1, dg0, dg1, wst, wnt, b, o):
    scale = _scale_from_deg(dg0, dg1)
    o[...] = (
        jnp.dot(h0[...], wst[0:128, :], precision=_HIGH)
        + jnp.dot(h1[...], wst[128:256, :], precision=_HIGH)
        + jnp.dot(ns0[0] * scale, wnt[0:128, :], precision=_HIGH)
        + jnp.dot(ns1[0] * scale, wnt[128:256, :], precision=_HIGH)
        + b[...]
    )


def _combine(h0, h1, ns, dg, wst, wnt, b):
    return pl.pallas_call(
        _combine_body,
        grid=_GRID,
        in_specs=[
            _row_spec(128), _row_spec(128),
            _half_spec(0, 128), _half_spec(1, 128),
            _half_spec(0, 128), _half_spec(1, 128),
            _full_spec(256, 256), _full_spec(256, 256), _full_spec(1, 256),
        ],
        out_specs=[_row_spec(128), _row_spec(128)],
        out_shape=[jax.ShapeDtypeStruct((N, 128), jnp.float32)] * 2,
    )(h0, h1, ns, ns, dg, dg, wst, wnt, b)


def _final(h0, h1, ns, dg, wst, wnt, b):
    return pl.pallas_call(
        _final_body,
        grid=_GRID,
        in_specs=[
            _row_spec(128), _row_spec(128),
            _half_spec(0, 128), _half_spec(1, 128),
            _half_spec(0, 128), _half_spec(1, 128),
            _full_spec(256, 64), _full_spec(256, 64), _full_spec(1, 64),
        ],
        out_specs=_row_spec(64),
        out_shape=jax.ShapeDtypeStruct((N, 64), jnp.float32),
    )(h0, h1, ns, ns, dg, dg, wst, wnt, b)


def kernel(x, edge_index, W_neigh0, W_self0, b0, W_neigh1, W_self1, b1,
           W_neigh2, W_self2, b2):
    src = edge_index[0]
    dst = edge_index[1]
    pad = jnp.arange(EP - E, dtype=jnp.int32)
    srcp = jnp.concatenate([src, pad % N]).reshape(NS, CH, LANE)
    dstp = jnp.concatenate(
        [dst, DUMMY_ROW + pad % (ACC_ROWS - N)]).reshape(NS, CH, LANE)

    zeros_w = jnp.zeros((ACC_ROWS, 128), jnp.float32)
    ones_w = jnp.ones((LANE, 128), jnp.float32)

    x0 = x[:, 0:128]
    x1 = x[:, 128:256]

    wst0 = W_self0.T
    wnt0 = W_neigh0.T
    wst1 = W_self1.T
    wnt1 = W_neigh1.T
    wnt2 = W_neigh2.T
    wst2 = W_self2.T
    b0r = b0.reshape(1, D_H)
    b1r = b1.reshape(1, D_H)
    b2r = b2.reshape(1, D_OUT)

    deg_f, = _deg(dstp, zeros_w, ones_w)
    dg = deg_f.reshape(NC, ACC_ROWS, 128)

    ns0_f, = _segsum(x0, x1, srcp, dstp, zeros_w)
    ns0 = ns0_f.reshape(NC, ACC_ROWS, 128)

    h10, h11 = _combine(x0, x1, ns0, dg, wst0, wnt0, b0r)

    ns1_f, = _segsum(h10, h11, srcp, dstp, zeros_w)
    ns1 = ns1_f.reshape(NC, ACC_ROWS, 128)

    h20, h21 = _combine(h10, h11, ns1, dg, wst1, wnt1, b1r)

    ns2_f, = _segsum(h20, h21, srcp, dstp, zeros_w)
    ns2 = ns2_f.reshape(NC, ACC_ROWS, 128)

    return _final(h20, h21, ns2, dg, wst2, wnt2, b2r)
